# baseline (speedup 1.0000x reference)
import functools

import jax
import jax.numpy as jnp
from jax import lax
from jax.experimental import pallas as pl
from jax.experimental.pallas import tpu as pltpu
from jax.experimental.pallas import tpu_sc as plsc

B = 16384
NF = 64
H1 = 256

NC = 2
NS = 16
NW = NC * NS
BPW = B // NW
CHUNK = 128
K = BPW // CHUNK


def _sc_gather(users3, items3, user_table, item_table):
    mesh = plsc.VectorSubcoreMesh(core_axis_name="c", subcore_axis_name="s")

    @functools.partial(
        pl.kernel,
        out_type=(
            jax.ShapeDtypeStruct((B, NF), jnp.float32),
            jax.ShapeDtypeStruct((B, NF), jnp.float32),
        ),
        mesh=mesh,
        scratch_types=[
            pltpu.VMEM((K, CHUNK), jnp.int32),
            pltpu.VMEM((K, CHUNK), jnp.int32),
            pltpu.VMEM((BPW, NF), jnp.float32),
            pltpu.VMEM((BPW, NF), jnp.float32),
            pltpu.SemaphoreType.DMA,
        ],
        compiler_params=pltpu.CompilerParams(use_tc_tiling_on_sc=False),
    )
    def k(users_hbm, items_hbm, ut_hbm, it_hbm, u_out, i_out,
          idx_u, idx_i, rows_u, rows_i, sem):
        wid = lax.axis_index("s") * NC + lax.axis_index("c")
        base = wid * BPW
        pltpu.sync_copy(users_hbm.at[wid], idx_u)
        pltpu.sync_copy(items_hbm.at[wid], idx_i)
        copies = []
        for j in range(K):
            copies.append(pltpu.async_copy(
                ut_hbm.at[idx_u.at[j]], rows_u.at[pl.ds(j * CHUNK, CHUNK)], sem))
            copies.append(pltpu.async_copy(
                it_hbm.at[idx_i.at[j]], rows_i.at[pl.ds(j * CHUNK, CHUNK)], sem))
        for c in copies:
            c.wait()
        pltpu.sync_copy(rows_u, u_out.at[pl.ds(base, BPW)])
        pltpu.sync_copy(rows_i, i_out.at[pl.ds(base, BPW)])

    return k(users3, items3, user_table, item_table)


BS = 2048


def _mlp_body(u_ref, i_ref, w1u_ref, w1i_ref, b1_ref, w2t_ref, b2_ref, o_ref):
    h = (
        jnp.dot(u_ref[...], w1u_ref[...], preferred_element_type=jnp.float32)
        + jnp.dot(i_ref[...], w1i_ref[...], preferred_element_type=jnp.float32)
        + b1_ref[...]
    )
    h = jnp.maximum(h, 0.0)
    o_ref[...] = jnp.sum(h * w2t_ref[...], axis=1, keepdims=True) + b2_ref[...]


def _mlp(u_emb, i_emb, W1u, W1i, b1, W2t, b2):
    return pl.pallas_call(
        _mlp_body,
        grid=(B // BS,),
        in_specs=[
            pl.BlockSpec((BS, NF), lambda i: (i, 0)),
            pl.BlockSpec((BS, NF), lambda i: (i, 0)),
            pl.BlockSpec((NF, H1), lambda i: (0, 0)),
            pl.BlockSpec((NF, H1), lambda i: (0, 0)),
            pl.BlockSpec((1, H1), lambda i: (0, 0)),
            pl.BlockSpec((1, H1), lambda i: (0, 0)),
            pl.BlockSpec((1, 1), lambda i: (0, 0)),
        ],
        out_specs=pl.BlockSpec((BS, 1), lambda i: (i, 0)),
        out_shape=jax.ShapeDtypeStruct((B, 1), jnp.float32),
    )(u_emb, i_emb, W1u, W1i, b1, W2t, b2)


@jax.jit
def kernel(users, items, user_table, item_table, W1, b1, W2, b2):
    users3 = users.reshape(NW, K, CHUNK)
    items3 = items.reshape(NW, K, CHUNK)
    u_emb, i_emb = _sc_gather(users3, items3, user_table, item_table)
    W1u = W1[:NF]
    W1i = W1[NF:]
    return _mlp(u_emb, i_emb, W1u, W1i,
                b1.reshape(1, H1), W2.reshape(1, H1), b2.reshape(1, 1))
